# baseline (speedup 1.0000x reference)
import functools

import jax
import jax.numpy as jnp
from jax import lax
from jax.experimental import pallas as pl
from jax.experimental.pallas import tpu as pltpu
from jax.experimental.pallas import tpu_sc as plsc

EMBED = 100
EPAD = 128
BATCH = 16384
NROWS = 100000

NC = 2
NS = 16
NW = NC * NS
LANES = 16
BPW = BATCH // NW
CHUNK = 128
NCHUNK = BPW // CHUNK

REPACK_ROWS = 5000


def _tc_repack(user_emb, recipe_emb):
    def body(u_ref, r_ref, uo_ref, ro_ref):
        zpad = jnp.zeros((REPACK_ROWS, EPAD - EMBED), jnp.float32)
        uo_ref[...] = jnp.concatenate([u_ref[...], zpad], axis=-1)
        ro_ref[...] = jnp.concatenate([r_ref[...], zpad], axis=-1)

    grid = NROWS // REPACK_ROWS
    spec_in = pl.BlockSpec((REPACK_ROWS, EMBED), lambda i: (i, 0))
    spec_out = pl.BlockSpec((REPACK_ROWS, EPAD), lambda i: (i, 0))
    return pl.pallas_call(
        body,
        grid=(grid,),
        in_specs=[spec_in, spec_in],
        out_specs=[spec_out, spec_out],
        out_shape=[jax.ShapeDtypeStruct((NROWS, EPAD), jnp.float32)] * 2,
    )(user_emb, recipe_emb)


def _sc_gather_contract(user_emb, recipe_emb, cat_emb, ubias, rbias,
                        uidx_a, ridx_a, cidx_a):
    mesh = plsc.VectorSubcoreMesh(core_axis_name="c", subcore_axis_name="s")

    @functools.partial(
        pl.kernel,
        out_type=[
            jax.ShapeDtypeStruct((NW, LANES), jnp.float32),
            jax.ShapeDtypeStruct((BATCH,), jnp.float32),
        ],
        mesh=mesh,
        compiler_params=pltpu.CompilerParams(use_tc_tiling_on_sc=True),
        scratch_types=[
            pltpu.VMEM((NCHUNK, CHUNK), jnp.int32),
            pltpu.VMEM((NCHUNK, CHUNK), jnp.int32),
            pltpu.VMEM((NCHUNK, CHUNK), jnp.int32),
            pltpu.VMEM((CHUNK, EPAD), jnp.float32),
            pltpu.VMEM((CHUNK, EPAD), jnp.float32),
            pltpu.VMEM((CHUNK, EPAD), jnp.float32),
            pltpu.VMEM((NCHUNK, CHUNK), jnp.float32),
            pltpu.VMEM((NCHUNK, CHUNK), jnp.float32),
            pltpu.VMEM((NCHUNK, CHUNK), jnp.float32),
            pltpu.VMEM((LANES,), jnp.float32),
            pltpu.SemaphoreType.DMA,
            pltpu.SemaphoreType.DMA,
        ],
    )
    def body(uemb_h, remb_h, cemb_h, ub_h, rb_h, ui_h, ri_h, ci_h,
             part_h, bsum_h,
             uidx, ridx, cidx, urows, rrows, crows, ub, rb, bsum, accv,
             semg, semb):
        wid = lax.axis_index("s") * NC + lax.axis_index("c")
        base = wid * BPW

        for j in range(NCHUNK):
            off = pl.ds(base + j * CHUNK, CHUNK)
            pltpu.sync_copy(ui_h.at[off], uidx.at[j])
            pltpu.sync_copy(ri_h.at[off], ridx.at[j])
            pltpu.sync_copy(ci_h.at[off], cidx.at[j])

        bias_cps = []
        for j in range(NCHUNK):
            bias_cps.append(pltpu.async_copy(ub_h.at[uidx.at[j]], ub.at[j], semb))
            bias_cps.append(pltpu.async_copy(rb_h.at[ridx.at[j]], rb.at[j], semb))

        acc = jnp.zeros((LANES,), jnp.float32)
        for j in range(NCHUNK):
            cu = pltpu.async_copy(uemb_h.at[uidx.at[j]], urows, semg)
            cr = pltpu.async_copy(remb_h.at[ridx.at[j]], rrows, semg)
            cc = pltpu.async_copy(cemb_h.at[cidx.at[j]], crows, semg)
            cu.wait(); cr.wait(); cc.wait()

            def row_body(i, a, _u=urows, _r=rrows, _c=crows):
                for t in range(7):
                    sl = pl.ds(16 * t, LANES)
                    a = a + (_u[i, sl] + _c[i, sl]) * _r[i, sl]
                return a

            acc = lax.fori_loop(0, CHUNK, row_body, acc)

        for cp in bias_cps:
            cp.wait()
        for j in range(NCHUNK):
            for k in range(CHUNK // LANES):
                sl = pl.ds(k * LANES, LANES)
                bsum[j, sl] = ub[j, sl] + rb[j, sl]
            pltpu.sync_copy(bsum.at[j], bsum_h.at[pl.ds(base + j * CHUNK, CHUNK)])

        accv[...] = acc
        pltpu.sync_copy(accv, part_h.at[wid])

    return body(user_emb, recipe_emb, cat_emb, ubias, rbias,
                uidx_a, ridx_a, cidx_a)


def _tc_epilogue(partials, bias_sum_2d):
    def body(part_ref, bsum_ref, out_ref):
        s = jnp.sum(part_ref[...])
        out_ref[...] = jax.nn.sigmoid(bsum_ref[...] + s)

    return pl.pallas_call(
        body,
        out_shape=jax.ShapeDtypeStruct(bias_sum_2d.shape, jnp.float32),
    )(partials, bias_sum_2d)


def kernel(user_emb, user_bias_tbl, recipe_emb, recipe_bias_tbl, cat_emb, inputs):
    idx = inputs.astype(jnp.int32)
    uidx = idx[:, 0]
    ridx = idx[:, 1]
    cidx = idx[:, 2]
    ubias = user_bias_tbl.reshape(-1)
    rbias = recipe_bias_tbl.reshape(-1)

    pad = [(0, 0), (0, EPAD - EMBED)]
    ue = jnp.pad(user_emb, pad)
    re_ = jnp.pad(recipe_emb, pad)
    ce = jnp.pad(cat_emb, pad)

    partials, bias_sum = _sc_gather_contract(
        ue, re_, ce, ubias, rbias, uidx, ridx, cidx)

    out2d = _tc_epilogue(partials, bias_sum.reshape(128, 128))
    return out2d.reshape(BATCH, 1)


# speedup vs baseline: 2.1087x; 2.1087x over previous
import functools

import jax
import jax.numpy as jnp
from jax import lax
from jax.experimental import pallas as pl
from jax.experimental.pallas import tpu as pltpu
from jax.experimental.pallas import tpu_sc as plsc

EMBED = 100
EPAD = 128
BATCH = 16384
NROWS = 100000

NC = 2
NS = 16
NW = NC * NS
LANES = 16
BPW = BATCH // NW
CHUNK = 128
NCHUNK = BPW // CHUNK

REPACK_ROWS = 5000


def _tc_repack(user_emb, recipe_emb):
    def body(u_ref, r_ref, uo_ref, ro_ref):
        zpad = jnp.zeros((REPACK_ROWS, EPAD - EMBED), jnp.float32)
        uo_ref[...] = jnp.concatenate([u_ref[...], zpad], axis=-1)
        ro_ref[...] = jnp.concatenate([r_ref[...], zpad], axis=-1)

    grid = NROWS // REPACK_ROWS
    spec_in = pl.BlockSpec((REPACK_ROWS, EMBED), lambda i: (i, 0))
    spec_out = pl.BlockSpec((REPACK_ROWS, EPAD), lambda i: (i, 0))
    return pl.pallas_call(
        body,
        grid=(grid,),
        in_specs=[spec_in, spec_in],
        out_specs=[spec_out, spec_out],
        out_shape=[jax.ShapeDtypeStruct((NROWS, EPAD), jnp.float32)] * 2,
    )(user_emb, recipe_emb)


def _sc_gather_contract(user_emb, recipe_emb, cat_emb, ubias, rbias,
                        uidx_a, ridx_a, cidx_a):
    mesh = plsc.VectorSubcoreMesh(core_axis_name="c", subcore_axis_name="s")

    @functools.partial(
        pl.kernel,
        out_type=[
            jax.ShapeDtypeStruct((NW, LANES), jnp.float32),
            jax.ShapeDtypeStruct((BATCH,), jnp.float32),
        ],
        mesh=mesh,
        compiler_params=pltpu.CompilerParams(use_tc_tiling_on_sc=True),
        scratch_types=[
            pltpu.VMEM((NCHUNK, CHUNK), jnp.int32),
            pltpu.VMEM((NCHUNK, CHUNK), jnp.int32),
            pltpu.VMEM((NCHUNK, CHUNK), jnp.int32),
            pltpu.VMEM((CHUNK, EPAD), jnp.float32),
            pltpu.VMEM((CHUNK, EPAD), jnp.float32),
            pltpu.VMEM((CHUNK, EPAD), jnp.float32),
            pltpu.VMEM((NCHUNK, CHUNK), jnp.float32),
            pltpu.VMEM((NCHUNK, CHUNK), jnp.float32),
            pltpu.VMEM((NCHUNK, CHUNK), jnp.float32),
            pltpu.VMEM((LANES,), jnp.float32),
            pltpu.SemaphoreType.DMA,
            pltpu.SemaphoreType.DMA,
        ],
    )
    def body(uemb_h, remb_h, cemb_h, ub_h, rb_h, ui_h, ri_h, ci_h,
             part_h, bsum_h,
             uidx, ridx, cidx, urows, rrows, crows, ub, rb, bsum, accv,
             semg, semb):
        wid = lax.axis_index("s") * NC + lax.axis_index("c")
        base = wid * BPW

        for j in range(NCHUNK):
            off = pl.ds(base + j * CHUNK, CHUNK)
            pltpu.sync_copy(ui_h.at[off], uidx.at[j])
            pltpu.sync_copy(ri_h.at[off], ridx.at[j])
            pltpu.sync_copy(ci_h.at[off], cidx.at[j])

        bias_cps = []
        for j in range(NCHUNK):
            bias_cps.append(pltpu.async_copy(ub_h.at[uidx.at[j]], ub.at[j], semb))
            bias_cps.append(pltpu.async_copy(rb_h.at[ridx.at[j]], rb.at[j], semb))

        acc = jnp.zeros((LANES,), jnp.float32)
        for j in range(NCHUNK):
            cu = pltpu.async_copy(uemb_h.at[uidx.at[j]], urows, semg)
            cr = pltpu.async_copy(remb_h.at[ridx.at[j]], rrows, semg)
            cc = pltpu.async_copy(cemb_h.at[cidx.at[j]], crows, semg)
            cu.wait(); cr.wait(); cc.wait()

            def row_body(i, a, _u=urows, _r=rrows, _c=crows):
                for t in range(7):
                    sl = pl.ds(16 * t, LANES)
                    a = a + (_u[i, sl] + _c[i, sl]) * _r[i, sl]
                return a

            acc = lax.fori_loop(0, CHUNK, row_body, acc)

        for cp in bias_cps:
            cp.wait()
        for j in range(NCHUNK):
            for k in range(CHUNK // LANES):
                sl = pl.ds(k * LANES, LANES)
                bsum[j, sl] = ub[j, sl] + rb[j, sl]
            pltpu.sync_copy(bsum.at[j], bsum_h.at[pl.ds(base + j * CHUNK, CHUNK)])

        accv[...] = acc
        pltpu.sync_copy(accv, part_h.at[wid])

    return body(user_emb, recipe_emb, cat_emb, ubias, rbias,
                uidx_a, ridx_a, cidx_a)


def _tc_epilogue(partials, bias_sum_2d):
    def body(part_ref, bsum_ref, out_ref):
        s = jnp.sum(part_ref[...])
        out_ref[...] = jax.nn.sigmoid(bsum_ref[...] + s)

    return pl.pallas_call(
        body,
        out_shape=jax.ShapeDtypeStruct(bias_sum_2d.shape, jnp.float32),
    )(partials, bias_sum_2d)


def kernel(user_emb, user_bias_tbl, recipe_emb, recipe_bias_tbl, cat_emb, inputs):
    idx = inputs.astype(jnp.int32)
    uidx = idx[:, 0]
    ridx = idx[:, 1]
    cidx = idx[:, 2]
    ubias = user_bias_tbl.reshape(-1)
    rbias = recipe_bias_tbl.reshape(-1)

    ue, re_ = _tc_repack(user_emb, recipe_emb)
    ce = jnp.pad(cat_emb, [(0, 0), (0, EPAD - EMBED)])

    partials, bias_sum = _sc_gather_contract(
        ue, re_, ce, ubias, rbias, uidx, ridx, cidx)

    out2d = _tc_epilogue(partials, bias_sum.reshape(128, 128))
    return out2d.reshape(BATCH, 1)
